# baseline (speedup 1.0000x reference)
import jax
import jax.numpy as jnp
from jax.experimental import pallas as pl
from jax.experimental.pallas import tpu as pltpu


def _gat_kernel(adj_ref, x_ref, W1_ref, as1_ref, ad1_ref, b1_ref,
                W2_ref, as2_ref, ad2_ref, b2_ref, out_ref):
    BB = adj_ref.shape[0]
    N = adj_ref.shape[1]
    for bb in range(BB):
        _gat_one(adj_ref, x_ref, W1_ref, as1_ref, ad1_ref, b1_ref,
                 W2_ref, as2_ref, ad2_ref, b2_ref, out_ref, bb, N)


def _gat_one(adj_ref, x_ref, W1_ref, as1_ref, ad1_ref, b1_ref,
             W2_ref, as2_ref, ad2_ref, b2_ref, out_ref, bb, N):
    adj = adj_ref[bb]
    x = x_ref[bb]

    ii = jax.lax.broadcasted_iota(jnp.int32, (N, N), 0)
    jj = jax.lax.broadcasted_iota(jnp.int32, (N, N), 1)
    diag = ii == jj
    live = (adj != 0) | diag
    mbias = jnp.where(live, 0.0, -1e30)
    ones_col = jnp.ones((N, 1), dtype=jnp.float32)

    def masked_attn(as_col, ad_row, h_head):
        h_aug = jnp.concatenate([h_head, ones_col], axis=1)
        raws = []
        for c in range(N // 128):
            sl = slice(c * 128, (c + 1) * 128)
            v = as_col + ad_row[:, sl]
            logits = jnp.maximum(v, 0.2 * v) + mbias[:, sl]
            amax = jnp.max(logits, axis=0, keepdims=True)
            ex = jnp.exp(logits - amax)
            raws.append(jax.lax.dot_general(
                ex, h_aug, (((0,), (0,)), ((), ())),
                preferred_element_type=jnp.float32))
        raw = jnp.concatenate(raws, axis=0)
        den = raw[:, -1:] + 1e-16
        return raw[:, :-1] / den

    h1 = jnp.dot(x, W1_ref[...], preferred_element_type=jnp.float32)
    as1 = as1_ref[0]
    ad1 = ad1_ref[0]
    outs = []
    for hd in range(4):
        h_head = h1[:, hd * 32:(hd + 1) * 32]
        a_s = as1[hd:hd + 1, :]
        a_d = ad1[hd:hd + 1, :]
        as_col = jax.lax.dot_general(
            h_head, a_s, (((1,), (1,)), ((), ())),
            preferred_element_type=jnp.float32)
        ad_row = jax.lax.dot_general(
            a_d, h_head, (((1,), (1,)), ((), ())),
            preferred_element_type=jnp.float32)
        outs.append(masked_attn(as_col, ad_row, h_head))
    g1 = jnp.concatenate(outs, axis=1) + b1_ref[...].reshape(1, -1)
    g1 = jnp.where(g1 > 0, g1, jnp.exp(jnp.minimum(g1, 0.0)) - 1.0)

    h2 = jnp.dot(g1, W2_ref[...], preferred_element_type=jnp.float32)
    as_col2 = jax.lax.dot_general(
        h2, as2_ref[0], (((1,), (1,)), ((), ())),
        preferred_element_type=jnp.float32)
    ad_row2 = jax.lax.dot_general(
        ad2_ref[0], h2, (((1,), (1,)), ((), ())),
        preferred_element_type=jnp.float32)
    out_ref[bb] = masked_attn(as_col2, ad_row2, h2) + b2_ref[...].reshape(1, -1)


@jax.jit
def kernel(adj, x, W1, a_src1, a_dst1, b1, W2, a_src2, a_dst2, b2):
    B, N, _ = adj.shape
    in_dim = x.shape[-1]
    heads, hidden = a_src1.shape[1], a_src1.shape[2]
    out_dim = W2.shape[1]

    BB = 1
    out = pl.pallas_call(
        _gat_kernel,
        grid=(B // BB,),
        in_specs=[
            pl.BlockSpec((BB, N, N), lambda b: (b, 0, 0)),
            pl.BlockSpec((BB, N, in_dim), lambda b: (b, 0, 0)),
            pl.BlockSpec((in_dim, heads * hidden), lambda b: (0, 0)),
            pl.BlockSpec((1, heads, hidden), lambda b: (0, 0, 0)),
            pl.BlockSpec((1, heads, hidden), lambda b: (0, 0, 0)),
            pl.BlockSpec((heads * hidden,), lambda b: (0,)),
            pl.BlockSpec((heads * hidden, out_dim), lambda b: (0, 0)),
            pl.BlockSpec((1, 1, out_dim), lambda b: (0, 0, 0)),
            pl.BlockSpec((1, 1, out_dim), lambda b: (0, 0, 0)),
            pl.BlockSpec((out_dim,), lambda b: (0,)),
        ],
        out_specs=pl.BlockSpec((BB, N, out_dim), lambda b: (b, 0, 0)),
        out_shape=jax.ShapeDtypeStruct((B, N, out_dim), jnp.float32),
        compiler_params=pltpu.CompilerParams(
            dimension_semantics=("parallel",)),
    )(adj, x, W1, a_src1, a_dst1, b1, W2, a_src2, a_dst2, b2)
    return out


# speedup vs baseline: 1.0812x; 1.0812x over previous
import jax
import jax.numpy as jnp
from jax.experimental import pallas as pl
from jax.experimental.pallas import tpu as pltpu


def _gat_kernel(adj_ref, x_ref, W1_ref, as1_ref, ad1_ref, b1_ref,
                W2_ref, as2_ref, ad2_ref, b2_ref, out_ref):
    BB = adj_ref.shape[0]
    N = adj_ref.shape[1]
    for bb in range(BB):
        _gat_one(adj_ref, x_ref, W1_ref, as1_ref, ad1_ref, b1_ref,
                 W2_ref, as2_ref, ad2_ref, b2_ref, out_ref, bb, N)


def _gat_one(adj_ref, x_ref, W1_ref, as1_ref, ad1_ref, b1_ref,
             W2_ref, as2_ref, ad2_ref, b2_ref, out_ref, bb, N):
    adj = adj_ref[bb]
    x = x_ref[bb]

    ii = jax.lax.broadcasted_iota(jnp.int32, (N, N), 0)
    jj = jax.lax.broadcasted_iota(jnp.int32, (N, N), 1)
    diag = ii == jj
    live = (adj != 0) | diag
    liveF = jnp.where(live, 1.0, 0.0)
    ones_col = jnp.ones((N, 1), dtype=jnp.float32)

    def masked_attn(as_col, ad_row, h_head):
        gmax = jnp.max(as_col)
        t = gmax + ad_row
        bound = jnp.maximum(t, 0.2 * t)
        ea1 = jnp.exp(as_col - gmax)
        ea2 = jnp.exp(0.2 * (as_col - gmax))
        eb1 = jnp.exp(ad_row + gmax - bound)
        eb2 = jnp.exp(0.2 * (ad_row + gmax) - bound)
        pos = (as_col + ad_row) >= 0
        ex = jnp.where(pos, ea1 * eb1, ea2 * eb2) * liveF
        h_aug = jnp.concatenate([h_head, ones_col], axis=1)
        raw = jax.lax.dot_general(
            ex, h_aug, (((0,), (0,)), ((), ())),
            preferred_element_type=jnp.float32)
        den = raw[:, -1:] + 1e-16
        return raw[:, :-1] / den

    h1 = jnp.dot(x, W1_ref[...], preferred_element_type=jnp.float32)
    as1 = as1_ref[0]
    ad1 = ad1_ref[0]
    outs = []
    for hd in range(4):
        h_head = h1[:, hd * 32:(hd + 1) * 32]
        a_s = as1[hd:hd + 1, :]
        a_d = ad1[hd:hd + 1, :]
        as_col = jax.lax.dot_general(
            h_head, a_s, (((1,), (1,)), ((), ())),
            preferred_element_type=jnp.float32)
        ad_row = jax.lax.dot_general(
            a_d, h_head, (((1,), (1,)), ((), ())),
            preferred_element_type=jnp.float32)
        outs.append(masked_attn(as_col, ad_row, h_head))
    g1 = jnp.concatenate(outs, axis=1) + b1_ref[...].reshape(1, -1)
    g1 = jnp.where(g1 > 0, g1, jnp.exp(jnp.minimum(g1, 0.0)) - 1.0)

    h2 = jnp.dot(g1, W2_ref[...], preferred_element_type=jnp.float32)
    as_col2 = jax.lax.dot_general(
        h2, as2_ref[0], (((1,), (1,)), ((), ())),
        preferred_element_type=jnp.float32)
    ad_row2 = jax.lax.dot_general(
        ad2_ref[0], h2, (((1,), (1,)), ((), ())),
        preferred_element_type=jnp.float32)
    out_ref[bb] = masked_attn(as_col2, ad_row2, h2) + b2_ref[...].reshape(1, -1)


@jax.jit
def kernel(adj, x, W1, a_src1, a_dst1, b1, W2, a_src2, a_dst2, b2):
    B, N, _ = adj.shape
    in_dim = x.shape[-1]
    heads, hidden = a_src1.shape[1], a_src1.shape[2]
    out_dim = W2.shape[1]

    BB = 1
    out = pl.pallas_call(
        _gat_kernel,
        grid=(B // BB,),
        in_specs=[
            pl.BlockSpec((BB, N, N), lambda b: (b, 0, 0)),
            pl.BlockSpec((BB, N, in_dim), lambda b: (b, 0, 0)),
            pl.BlockSpec((in_dim, heads * hidden), lambda b: (0, 0)),
            pl.BlockSpec((1, heads, hidden), lambda b: (0, 0, 0)),
            pl.BlockSpec((1, heads, hidden), lambda b: (0, 0, 0)),
            pl.BlockSpec((heads * hidden,), lambda b: (0,)),
            pl.BlockSpec((heads * hidden, out_dim), lambda b: (0, 0)),
            pl.BlockSpec((1, 1, out_dim), lambda b: (0, 0, 0)),
            pl.BlockSpec((1, 1, out_dim), lambda b: (0, 0, 0)),
            pl.BlockSpec((out_dim,), lambda b: (0,)),
        ],
        out_specs=pl.BlockSpec((BB, N, out_dim), lambda b: (b, 0, 0)),
        out_shape=jax.ShapeDtypeStruct((B, N, out_dim), jnp.float32),
        compiler_params=pltpu.CompilerParams(
            dimension_semantics=("parallel",)),
    )(adj, x, W1, a_src1, a_dst1, b1, W2, a_src2, a_dst2, b2)
    return out


# speedup vs baseline: 1.1367x; 1.0513x over previous
import jax
import jax.numpy as jnp
from jax.experimental import pallas as pl
from jax.experimental.pallas import tpu as pltpu


def _gat_kernel(adj_ref, x_ref, W1_ref, as1_ref, ad1_ref, b1_ref,
                W2_ref, as2_ref, ad2_ref, b2_ref, out_ref):
    BB = adj_ref.shape[0]
    N = adj_ref.shape[1]
    for bb in range(BB):
        _gat_one(adj_ref, x_ref, W1_ref, as1_ref, ad1_ref, b1_ref,
                 W2_ref, as2_ref, ad2_ref, b2_ref, out_ref, bb, N)


def _gat_one(adj_ref, x_ref, W1_ref, as1_ref, ad1_ref, b1_ref,
             W2_ref, as2_ref, ad2_ref, b2_ref, out_ref, bb, N):
    adj = adj_ref[bb]
    x = x_ref[bb]

    ii = jax.lax.broadcasted_iota(jnp.int32, (N, N), 0)
    jj = jax.lax.broadcasted_iota(jnp.int32, (N, N), 1)
    diag = ii == jj
    live = (adj != 0) | diag
    liveF = jnp.where(live, 1.0, 0.0)
    ones_col = jnp.ones((N, 1), dtype=jnp.float32)

    def masked_attn(as_col, ad_row, h_head):
        gmax = jnp.max(as_col)
        t = gmax + ad_row
        bound = jnp.maximum(t, 0.2 * t)
        ea1 = jnp.exp(as_col - gmax)
        ea2 = jnp.exp(0.2 * (as_col - gmax))
        eb1 = jnp.exp(ad_row + gmax - bound)
        eb2 = jnp.exp(0.2 * (ad_row + gmax) - bound)
        ex = jnp.maximum(ea1 * eb1, ea2 * eb2) * liveF
        h_aug = jnp.concatenate([h_head, ones_col], axis=1)
        raw = jax.lax.dot_general(
            ex, h_aug, (((0,), (0,)), ((), ())),
            preferred_element_type=jnp.float32)
        den = raw[:, -1:] + 1e-16
        return raw[:, :-1] / den

    h1 = jnp.dot(x, W1_ref[...], preferred_element_type=jnp.float32)
    as1 = as1_ref[0]
    ad1 = ad1_ref[0]
    outs = []
    for hd in range(4):
        h_head = h1[:, hd * 32:(hd + 1) * 32]
        a_s = as1[hd:hd + 1, :]
        a_d = ad1[hd:hd + 1, :]
        as_col = jax.lax.dot_general(
            h_head, a_s, (((1,), (1,)), ((), ())),
            preferred_element_type=jnp.float32)
        ad_row = jax.lax.dot_general(
            a_d, h_head, (((1,), (1,)), ((), ())),
            preferred_element_type=jnp.float32)
        outs.append(masked_attn(as_col, ad_row, h_head))
    g1 = jnp.concatenate(outs, axis=1) + b1_ref[...].reshape(1, -1)
    g1 = jnp.where(g1 > 0, g1, jnp.exp(jnp.minimum(g1, 0.0)) - 1.0)

    h2 = jnp.dot(g1, W2_ref[...], preferred_element_type=jnp.float32)
    as_col2 = jax.lax.dot_general(
        h2, as2_ref[0], (((1,), (1,)), ((), ())),
        preferred_element_type=jnp.float32)
    ad_row2 = jax.lax.dot_general(
        ad2_ref[0], h2, (((1,), (1,)), ((), ())),
        preferred_element_type=jnp.float32)
    out_ref[bb] = masked_attn(as_col2, ad_row2, h2) + b2_ref[...].reshape(1, -1)


@jax.jit
def kernel(adj, x, W1, a_src1, a_dst1, b1, W2, a_src2, a_dst2, b2):
    B, N, _ = adj.shape
    in_dim = x.shape[-1]
    heads, hidden = a_src1.shape[1], a_src1.shape[2]
    out_dim = W2.shape[1]

    BB = 1
    out = pl.pallas_call(
        _gat_kernel,
        grid=(B // BB,),
        in_specs=[
            pl.BlockSpec((BB, N, N), lambda b: (b, 0, 0)),
            pl.BlockSpec((BB, N, in_dim), lambda b: (b, 0, 0)),
            pl.BlockSpec((in_dim, heads * hidden), lambda b: (0, 0)),
            pl.BlockSpec((1, heads, hidden), lambda b: (0, 0, 0)),
            pl.BlockSpec((1, heads, hidden), lambda b: (0, 0, 0)),
            pl.BlockSpec((heads * hidden,), lambda b: (0,)),
            pl.BlockSpec((heads * hidden, out_dim), lambda b: (0, 0)),
            pl.BlockSpec((1, 1, out_dim), lambda b: (0, 0, 0)),
            pl.BlockSpec((1, 1, out_dim), lambda b: (0, 0, 0)),
            pl.BlockSpec((out_dim,), lambda b: (0,)),
        ],
        out_specs=pl.BlockSpec((BB, N, out_dim), lambda b: (b, 0, 0)),
        out_shape=jax.ShapeDtypeStruct((B, N, out_dim), jnp.float32),
        compiler_params=pltpu.CompilerParams(
            dimension_semantics=("parallel",)),
    )(adj, x, W1, a_src1, a_dst1, b1, W2, a_src2, a_dst2, b2)
    return out


# speedup vs baseline: 1.1469x; 1.0090x over previous
import jax
import jax.numpy as jnp
from jax.experimental import pallas as pl
from jax.experimental.pallas import tpu as pltpu


def _gat_kernel(adj_ref, x_ref, W1_ref, as1_ref, ad1_ref, b1_ref,
                W2_ref, as2_ref, ad2_ref, b2_ref, out_ref):
    BB = adj_ref.shape[0]
    N = adj_ref.shape[1]
    for bb in range(BB):
        _gat_one(adj_ref, x_ref, W1_ref, as1_ref, ad1_ref, b1_ref,
                 W2_ref, as2_ref, ad2_ref, b2_ref, out_ref, bb, N)


def _gat_one(adj_ref, x_ref, W1_ref, as1_ref, ad1_ref, b1_ref,
             W2_ref, as2_ref, ad2_ref, b2_ref, out_ref, bb, N):
    adj = adj_ref[bb]
    x = x_ref[bb]

    ii = jax.lax.broadcasted_iota(jnp.int32, (N, N), 0)
    jj = jax.lax.broadcasted_iota(jnp.int32, (N, N), 1)
    diag = ii == jj
    live = (adj != 0) | diag
    liveF = jnp.where(live, 1.0, 0.0)
    ones_col = jnp.ones((N, 1), dtype=jnp.float32)

    def masked_attn(as_col, ad_row, h_head):
        gmax = jnp.max(as_col)
        t = gmax + ad_row
        bound = jnp.maximum(t, 0.2 * t)
        ea1 = jnp.exp(as_col - gmax)
        ea2 = jnp.exp(0.2 * (as_col - gmax))
        eb1 = jnp.exp(ad_row + gmax - bound)
        eb2 = jnp.exp(0.2 * (ad_row + gmax) - bound)
        ex = jnp.maximum(ea1 * eb1, ea2 * eb2) * liveF
        h_aug = jnp.concatenate([h_head, ones_col], axis=1)
        raw = jax.lax.dot_general(
            ex.astype(jnp.bfloat16), h_aug.astype(jnp.bfloat16),
            (((0,), (0,)), ((), ())),
            preferred_element_type=jnp.float32)
        den = raw[:, -1:] + 1e-16
        return raw[:, :-1] / den

    h1 = jnp.dot(x, W1_ref[...], preferred_element_type=jnp.float32)
    as1 = as1_ref[0]
    ad1 = ad1_ref[0]
    outs = []
    for hd in range(4):
        h_head = h1[:, hd * 32:(hd + 1) * 32]
        a_s = as1[hd:hd + 1, :]
        a_d = ad1[hd:hd + 1, :]
        as_col = jax.lax.dot_general(
            h_head, a_s, (((1,), (1,)), ((), ())),
            preferred_element_type=jnp.float32)
        ad_row = jax.lax.dot_general(
            a_d, h_head, (((1,), (1,)), ((), ())),
            preferred_element_type=jnp.float32)
        outs.append(masked_attn(as_col, ad_row, h_head))
    g1 = jnp.concatenate(outs, axis=1) + b1_ref[...].reshape(1, -1)
    g1 = jnp.where(g1 > 0, g1, jnp.exp(jnp.minimum(g1, 0.0)) - 1.0)

    h2 = jnp.dot(g1, W2_ref[...], preferred_element_type=jnp.float32)
    as_col2 = jax.lax.dot_general(
        h2, as2_ref[0], (((1,), (1,)), ((), ())),
        preferred_element_type=jnp.float32)
    ad_row2 = jax.lax.dot_general(
        ad2_ref[0], h2, (((1,), (1,)), ((), ())),
        preferred_element_type=jnp.float32)
    out_ref[bb] = masked_attn(as_col2, ad_row2, h2) + b2_ref[...].reshape(1, -1)


@jax.jit
def kernel(adj, x, W1, a_src1, a_dst1, b1, W2, a_src2, a_dst2, b2):
    B, N, _ = adj.shape
    in_dim = x.shape[-1]
    heads, hidden = a_src1.shape[1], a_src1.shape[2]
    out_dim = W2.shape[1]

    BB = 1
    out = pl.pallas_call(
        _gat_kernel,
        grid=(B // BB,),
        in_specs=[
            pl.BlockSpec((BB, N, N), lambda b: (b, 0, 0)),
            pl.BlockSpec((BB, N, in_dim), lambda b: (b, 0, 0)),
            pl.BlockSpec((in_dim, heads * hidden), lambda b: (0, 0)),
            pl.BlockSpec((1, heads, hidden), lambda b: (0, 0, 0)),
            pl.BlockSpec((1, heads, hidden), lambda b: (0, 0, 0)),
            pl.BlockSpec((heads * hidden,), lambda b: (0,)),
            pl.BlockSpec((heads * hidden, out_dim), lambda b: (0, 0)),
            pl.BlockSpec((1, 1, out_dim), lambda b: (0, 0, 0)),
            pl.BlockSpec((1, 1, out_dim), lambda b: (0, 0, 0)),
            pl.BlockSpec((out_dim,), lambda b: (0,)),
        ],
        out_specs=pl.BlockSpec((BB, N, out_dim), lambda b: (b, 0, 0)),
        out_shape=jax.ShapeDtypeStruct((B, N, out_dim), jnp.float32),
        compiler_params=pltpu.CompilerParams(
            dimension_semantics=("parallel",)),
    )(adj, x, W1, a_src1, a_dst1, b1, W2, a_src2, a_dst2, b2)
    return out


# speedup vs baseline: 1.1946x; 1.0415x over previous
import jax
import jax.numpy as jnp
from jax.experimental import pallas as pl
from jax.experimental.pallas import tpu as pltpu


def _gat_kernel(adj_ref, x_ref, W1_ref, as1_ref, ad1_ref, b1_ref,
                W2_ref, as2_ref, ad2_ref, b2_ref, out_ref):
    BB = adj_ref.shape[0]
    N = adj_ref.shape[1]
    for bb in range(BB):
        _gat_one(adj_ref, x_ref, W1_ref, as1_ref, ad1_ref, b1_ref,
                 W2_ref, as2_ref, ad2_ref, b2_ref, out_ref, bb, N)


def _gat_one(adj_ref, x_ref, W1_ref, as1_ref, ad1_ref, b1_ref,
             W2_ref, as2_ref, ad2_ref, b2_ref, out_ref, bb, N):
    adj = adj_ref[bb]
    x = x_ref[bb]

    ii = jax.lax.broadcasted_iota(jnp.int32, (N, N), 0)
    jj = jax.lax.broadcasted_iota(jnp.int32, (N, N), 1)
    diag = ii == jj
    live = (adj != 0) | diag
    liveF = jnp.where(live, 1.0, 0.0)
    ones_col = jnp.ones((N, 1), dtype=jnp.float32)

    def masked_attn(as_col, ad_row, h_head):
        gmax = jnp.max(as_col)
        t = gmax + ad_row
        bound = jnp.maximum(t, 0.2 * t)
        ea1 = jnp.exp(as_col - gmax)
        ea2 = jnp.exp(0.2 * (as_col - gmax))
        eb1 = jnp.exp(ad_row + gmax - bound)
        eb2 = jnp.exp(0.2 * (ad_row + gmax) - bound)
        ex = jnp.maximum(ea1 * eb1, ea2 * eb2) * liveF
        h_aug = jnp.concatenate([h_head, ones_col], axis=1)
        raw = jax.lax.dot_general(
            ex.astype(jnp.bfloat16), h_aug.astype(jnp.bfloat16),
            (((0,), (0,)), ((), ())),
            preferred_element_type=jnp.float32)
        den = raw[:, -1:] + 1e-16
        return raw[:, :-1] / den

    h1 = jnp.dot(x, W1_ref[...], preferred_element_type=jnp.float32)
    as1 = as1_ref[0]
    ad1 = ad1_ref[0]
    outs = []
    for hd in range(4):
        h_head = h1[:, hd * 32:(hd + 1) * 32]
        a_s = as1[hd:hd + 1, :]
        a_d = ad1[hd:hd + 1, :]
        as_col = jax.lax.dot_general(
            h_head, a_s, (((1,), (1,)), ((), ())),
            preferred_element_type=jnp.float32)
        ad_row = jax.lax.dot_general(
            a_d, h_head, (((1,), (1,)), ((), ())),
            preferred_element_type=jnp.float32)
        outs.append(masked_attn(as_col, ad_row, h_head))
    g1 = jnp.concatenate(outs, axis=1) + b1_ref[...].reshape(1, -1)
    g1 = jnp.where(g1 > 0, g1, jnp.exp(jnp.minimum(g1, 0.0)) - 1.0)

    h2 = jnp.dot(g1, W2_ref[...], preferred_element_type=jnp.float32)
    as_col2 = jax.lax.dot_general(
        h2, as2_ref[0], (((1,), (1,)), ((), ())),
        preferred_element_type=jnp.float32)
    ad_row2 = jax.lax.dot_general(
        ad2_ref[0], h2, (((1,), (1,)), ((), ())),
        preferred_element_type=jnp.float32)
    out_ref[bb] = masked_attn(as_col2, ad_row2, h2) + b2_ref[...].reshape(1, -1)


@jax.jit
def kernel(adj, x, W1, a_src1, a_dst1, b1, W2, a_src2, a_dst2, b2):
    B, N, _ = adj.shape
    in_dim = x.shape[-1]
    heads, hidden = a_src1.shape[1], a_src1.shape[2]
    out_dim = W2.shape[1]

    BB = 2
    out = pl.pallas_call(
        _gat_kernel,
        grid=(B // BB,),
        in_specs=[
            pl.BlockSpec((BB, N, N), lambda b: (b, 0, 0)),
            pl.BlockSpec((BB, N, in_dim), lambda b: (b, 0, 0)),
            pl.BlockSpec((in_dim, heads * hidden), lambda b: (0, 0)),
            pl.BlockSpec((1, heads, hidden), lambda b: (0, 0, 0)),
            pl.BlockSpec((1, heads, hidden), lambda b: (0, 0, 0)),
            pl.BlockSpec((heads * hidden,), lambda b: (0,)),
            pl.BlockSpec((heads * hidden, out_dim), lambda b: (0, 0)),
            pl.BlockSpec((1, 1, out_dim), lambda b: (0, 0, 0)),
            pl.BlockSpec((1, 1, out_dim), lambda b: (0, 0, 0)),
            pl.BlockSpec((out_dim,), lambda b: (0,)),
        ],
        out_specs=pl.BlockSpec((BB, N, out_dim), lambda b: (b, 0, 0)),
        out_shape=jax.ShapeDtypeStruct((B, N, out_dim), jnp.float32),
        compiler_params=pltpu.CompilerParams(
            dimension_semantics=("parallel",)),
    )(adj, x, W1, a_src1, a_dst1, b1, W2, a_src2, a_dst2, b2)
    return out


# speedup vs baseline: 1.2022x; 1.0064x over previous
import jax
import jax.numpy as jnp
from jax.experimental import pallas as pl
from jax.experimental.pallas import tpu as pltpu


def _gat_kernel(adj_ref, x_ref, W1_ref, as1_ref, ad1_ref, b1_ref,
                W2_ref, as2_ref, ad2_ref, b2_ref, out_ref):
    BB = adj_ref.shape[0]
    N = adj_ref.shape[1]
    for bb in range(BB):
        _gat_one(adj_ref, x_ref, W1_ref, as1_ref, ad1_ref, b1_ref,
                 W2_ref, as2_ref, ad2_ref, b2_ref, out_ref, bb, N)


def _gat_one(adj_ref, x_ref, W1_ref, as1_ref, ad1_ref, b1_ref,
             W2_ref, as2_ref, ad2_ref, b2_ref, out_ref, bb, N):
    adj = adj_ref[bb]
    x = x_ref[bb]

    ii = jax.lax.broadcasted_iota(jnp.int32, (N, N), 0)
    jj = jax.lax.broadcasted_iota(jnp.int32, (N, N), 1)
    diag = ii == jj
    live = (adj != 0) | diag
    liveF = jnp.where(live, 1.0, 0.0)
    ones_col = jnp.ones((N, 1), dtype=jnp.float32)

    def masked_attn(as_col, ad_row, h_head):
        gmax = jnp.max(as_col)
        t = gmax + ad_row
        bound = jnp.maximum(t, 0.2 * t)
        ea1 = jnp.exp(as_col - gmax)
        ea2 = jnp.exp(0.2 * (as_col - gmax))
        eb1 = jnp.exp(ad_row + gmax - bound)
        eb2 = jnp.exp(0.2 * (ad_row + gmax) - bound)
        ex = jnp.maximum(ea1 * eb1, ea2 * eb2) * liveF
        h_aug = jnp.concatenate([h_head, ones_col], axis=1)
        raw = jax.lax.dot_general(
            ex.astype(jnp.bfloat16), h_aug.astype(jnp.bfloat16),
            (((0,), (0,)), ((), ())),
            preferred_element_type=jnp.float32)
        den = raw[:, -1:] + 1e-16
        return raw[:, :-1] / den

    h1 = jnp.dot(x, W1_ref[...], preferred_element_type=jnp.float32)
    as1 = as1_ref[0]
    ad1 = ad1_ref[0]
    outs = []
    for hd in range(4):
        h_head = h1[:, hd * 32:(hd + 1) * 32]
        a_s = as1[hd:hd + 1, :]
        a_d = ad1[hd:hd + 1, :]
        as_col = jax.lax.dot_general(
            h_head, a_s, (((1,), (1,)), ((), ())),
            preferred_element_type=jnp.float32)
        ad_row = jax.lax.dot_general(
            a_d, h_head, (((1,), (1,)), ((), ())),
            preferred_element_type=jnp.float32)
        outs.append(masked_attn(as_col, ad_row, h_head))
    g1 = jnp.concatenate(outs, axis=1) + b1_ref[...].reshape(1, -1)
    g1 = jnp.where(g1 > 0, g1, jnp.exp(jnp.minimum(g1, 0.0)) - 1.0)

    h2 = jnp.dot(g1, W2_ref[...], preferred_element_type=jnp.float32)
    as_col2 = jax.lax.dot_general(
        h2, as2_ref[0], (((1,), (1,)), ((), ())),
        preferred_element_type=jnp.float32)
    ad_row2 = jax.lax.dot_general(
        ad2_ref[0], h2, (((1,), (1,)), ((), ())),
        preferred_element_type=jnp.float32)
    out_ref[bb] = masked_attn(as_col2, ad_row2, h2) + b2_ref[...].reshape(1, -1)


@jax.jit
def kernel(adj, x, W1, a_src1, a_dst1, b1, W2, a_src2, a_dst2, b2):
    B, N, _ = adj.shape
    in_dim = x.shape[-1]
    heads, hidden = a_src1.shape[1], a_src1.shape[2]
    out_dim = W2.shape[1]

    BB = 4
    out = pl.pallas_call(
        _gat_kernel,
        grid=(B // BB,),
        in_specs=[
            pl.BlockSpec((BB, N, N), lambda b: (b, 0, 0)),
            pl.BlockSpec((BB, N, in_dim), lambda b: (b, 0, 0)),
            pl.BlockSpec((in_dim, heads * hidden), lambda b: (0, 0)),
            pl.BlockSpec((1, heads, hidden), lambda b: (0, 0, 0)),
            pl.BlockSpec((1, heads, hidden), lambda b: (0, 0, 0)),
            pl.BlockSpec((heads * hidden,), lambda b: (0,)),
            pl.BlockSpec((heads * hidden, out_dim), lambda b: (0, 0)),
            pl.BlockSpec((1, 1, out_dim), lambda b: (0, 0, 0)),
            pl.BlockSpec((1, 1, out_dim), lambda b: (0, 0, 0)),
            pl.BlockSpec((out_dim,), lambda b: (0,)),
        ],
        out_specs=pl.BlockSpec((BB, N, out_dim), lambda b: (b, 0, 0)),
        out_shape=jax.ShapeDtypeStruct((B, N, out_dim), jnp.float32),
        compiler_params=pltpu.CompilerParams(
            dimension_semantics=("parallel",)),
    )(adj, x, W1, a_src1, a_dst1, b1, W2, a_src2, a_dst2, b2)
    return out
